# baseline (speedup 1.0000x reference)
import functools

import jax
import jax.numpy as jnp
from jax import lax
from jax.experimental import pallas as pl
from jax.experimental.pallas import tpu as pltpu
from jax.experimental.pallas import tpu_sc as plsc

SH_V, YU_V = 24, 40
SH_D, YU_D = 32, 32
OUT_D = SH_D + YU_D
NC, NS, L = 2, 16, 16
NW = NC * NS
BLK = 128
NSLOT = 4
GRP = BLK // L


def _make_sc_kernel(rows_per_w):
    n_blocks = rows_per_w // BLK
    mesh = plsc.VectorSubcoreMesh(
        core_axis_name="c", subcore_axis_name="s",
        num_cores=NC, num_subcores=NS)

    @functools.partial(
        pl.kernel,
        out_type=jax.ShapeDtypeStruct((NW, n_blocks, BLK * OUT_D),
                                      jnp.float32),
        mesh=mesh,
        scratch_types=[
            pltpu.VMEM((rows_per_w,), jnp.int32),
            pltpu.VMEM((rows_per_w,), jnp.int32),
            pltpu.VMEM((SH_V * SH_D,), jnp.float32),
            pltpu.VMEM((YU_V * YU_D,), jnp.float32),
            [pltpu.VMEM((BLK * OUT_D,), jnp.float32) for _ in range(NSLOT)],
            [pltpu.SemaphoreType.DMA for _ in range(NSLOT)],
            pltpu.SemaphoreType.DMA,
        ],
        compiler_params=pltpu.CompilerParams(needs_layout_passes=False),
    )
    def sc_kernel(sidx_hbm, yidx_hbm, sh_hbm, yu_hbm, out_hbm,
                  s_v, y_v, sh_v, yu_v, bufs, wsems, lsem):
        wid = lax.axis_index("s") * NC + lax.axis_index("c")

        pltpu.async_copy(sh_hbm, sh_v, lsem)
        pltpu.async_copy(yu_hbm, yu_v, lsem)
        pltpu.async_copy(sidx_hbm.at[wid], s_v, lsem)
        pltpu.async_copy(yidx_hbm.at[wid], y_v, lsem)
        pltpu.make_async_copy(sh_hbm, sh_v, lsem).wait()
        pltpu.make_async_copy(yu_hbm, yu_v, lsem).wait()
        pltpu.make_async_copy(sidx_hbm.at[wid], s_v, lsem).wait()
        pltpu.make_async_copy(yidx_hbm.at[wid], y_v, lsem).wait()

        dst0 = lax.iota(jnp.int32, L) * OUT_D

        def fill_group(b, g, buf):
            base = g * BLK + b * L
            src_s = s_v[pl.ds(base, L)] * SH_D
            src_y = y_v[pl.ds(base, L)] * YU_D
            dst = dst0 + b * (L * OUT_D)
            for j in range(SH_D):
                plsc.store_scatter(buf, [dst + j],
                                   plsc.load_gather(sh_v, [src_s + j]))
            for j in range(YU_D):
                plsc.store_scatter(buf, [dst + SH_D + j],
                                   plsc.load_gather(yu_v, [src_y + j]))

        def step(t, _):
            for r in range(NSLOT):
                g = t * NSLOT + r

                @pl.when(t > 0)
                def _reclaim():
                    pltpu.make_async_copy(
                        bufs[r], out_hbm.at[wid, g - NSLOT],
                        wsems[r]).wait()

                def fill(b, _):
                    fill_group(b, g, bufs[r])
                    return 0
                lax.fori_loop(0, GRP, fill, 0)
                pltpu.async_copy(bufs[r], out_hbm.at[wid, g], wsems[r])
            return 0

        lax.fori_loop(0, n_blocks // NSLOT, step, 0)

        for r in range(NSLOT):
            g = n_blocks - NSLOT + r
            pltpu.make_async_copy(bufs[r], out_hbm.at[wid, g],
                                  wsems[r]).wait()

    return sc_kernel


def kernel(shengmu_indices, yunmu_indices, shengmu_table, yunmu_table):
    batch, seq = shengmu_indices.shape
    n = batch * seq
    assert n % (NW * BLK * NSLOT) == 0
    rows_per_w = n // NW

    s = shengmu_indices.reshape(NW, rows_per_w)
    y = yunmu_indices.reshape(NW, rows_per_w)
    out = _make_sc_kernel(rows_per_w)(
        s, y, shengmu_table.reshape(-1), yunmu_table.reshape(-1))
    return out.reshape(batch, seq, OUT_D)


# speedup vs baseline: 2.7594x; 2.7594x over previous
import functools

import jax
import jax.numpy as jnp
from jax import lax
from jax.experimental import pallas as pl
from jax.experimental.pallas import tpu as pltpu
from jax.experimental.pallas import tpu_sc as plsc

SH_V, YU_V = 24, 40
SH_D, YU_D = 32, 32
OUT_D = SH_D + YU_D
TAB_ROWS = SH_V * YU_V
NC, NS, L = 2, 16, 16
NW = NC * NS
BLK = 128
NSLOT = 8
AHEAD = 4


def _build_table_body(sh_ref, yu_ref, out_ref):
    sh = sh_ref[...]
    yu = yu_ref[...]
    shb = jnp.broadcast_to(sh[:, None, :], (SH_V, YU_V, SH_D)).reshape(
        TAB_ROWS, SH_D)
    yub = jnp.broadcast_to(yu[None, :, :], (SH_V, YU_V, YU_D)).reshape(
        TAB_ROWS, YU_D)
    out_ref[...] = jnp.concatenate([shb, yub], axis=-1)


def _build_table(sh_table, yu_table):
    return pl.pallas_call(
        _build_table_body,
        out_shape=jax.ShapeDtypeStruct((TAB_ROWS, OUT_D), jnp.float32),
    )(sh_table, yu_table)


def _make_sc_kernel(n_blocks):
    mesh = plsc.VectorSubcoreMesh(
        core_axis_name="c", subcore_axis_name="s",
        num_cores=NC, num_subcores=NS)

    @functools.partial(
        pl.kernel,
        out_type=jax.ShapeDtypeStruct((NW, n_blocks, BLK, OUT_D), jnp.float32),
        mesh=mesh,
        scratch_types=[
            pltpu.VMEM((n_blocks, BLK), jnp.int32),
            pltpu.VMEM((n_blocks, BLK), jnp.int32),
            [pltpu.VMEM((BLK, OUT_D), jnp.float32) for _ in range(NSLOT)],
            [pltpu.SemaphoreType.DMA for _ in range(NSLOT)],
            [pltpu.SemaphoreType.DMA for _ in range(NSLOT)],
        ],
        compiler_params=pltpu.CompilerParams(use_tc_tiling_on_sc=False),
    )
    def sc_kernel(sidx_hbm, yidx_hbm, table_hbm, out_hbm,
                  comb_v, y_v, bufs, gsems, wsems):
        wid = lax.axis_index("s") * NC + lax.axis_index("c")

        pltpu.sync_copy(sidx_hbm.at[wid], comb_v)
        pltpu.sync_copy(yidx_hbm.at[wid], y_v)

        def fuse(t, _):
            i = t // (BLK // L)
            j = (t % (BLK // L)) * L
            comb_v[i, pl.ds(j, L)] = (
                comb_v[i, pl.ds(j, L)] * YU_V + y_v[i, pl.ds(j, L)])
            return 0
        lax.fori_loop(0, n_blocks * (BLK // L), fuse, 0)

        for h in range(AHEAD):
            pltpu.async_copy(table_hbm.at[comb_v.at[h]], bufs[h % NSLOT],
                             gsems[h % NSLOT])

        def step(t, _):
            for r in range(NSLOT):
                g = t * NSLOT + r
                pltpu.make_async_copy(table_hbm.at[comb_v.at[g]], bufs[r],
                                      gsems[r]).wait()
                pltpu.async_copy(bufs[r], out_hbm.at[wid, g], wsems[r])
                h = g + AHEAD
                s = (r + AHEAD) % NSLOT

                @pl.when(h < n_blocks)
                def _prefetch():
                    @pl.when(h >= NSLOT)
                    def _reclaim():
                        pltpu.make_async_copy(
                            bufs[s], out_hbm.at[wid, h - NSLOT],
                            wsems[s]).wait()
                    pltpu.async_copy(table_hbm.at[comb_v.at[h]], bufs[s],
                                     gsems[s])
            return 0

        lax.fori_loop(0, n_blocks // NSLOT, step, 0)

        for r in range(NSLOT):
            g = n_blocks - NSLOT + r
            pltpu.make_async_copy(bufs[r], out_hbm.at[wid, g],
                                  wsems[r]).wait()

    return sc_kernel


def kernel(shengmu_indices, yunmu_indices, shengmu_table, yunmu_table):
    batch, seq = shengmu_indices.shape
    n = batch * seq
    assert n % (NW * BLK * NSLOT) == 0
    n_blocks = n // (NW * BLK)

    table = _build_table(shengmu_table, yunmu_table)
    s = shengmu_indices.reshape(NW, n_blocks, BLK)
    y = yunmu_indices.reshape(NW, n_blocks, BLK)
    out = _make_sc_kernel(n_blocks)(s, y, table)
    return out.reshape(batch, seq, OUT_D)


# speedup vs baseline: 3.8823x; 1.4069x over previous
import functools

import jax
import jax.numpy as jnp
from jax import lax
from jax.experimental import pallas as pl
from jax.experimental.pallas import tpu as pltpu
from jax.experimental.pallas import tpu_sc as plsc

SH_V, YU_V = 24, 40
SH_D, YU_D = 32, 32
OUT_D = SH_D + YU_D
TAB_ROWS = SH_V * YU_V
NC, NS, L = 2, 16, 16
NW = NC * NS
BLK = 128
NSLOT = 8
AHEAD = 4


def _build_table_body(sh_ref, yu_ref, out_ref):
    sh = sh_ref[...]
    yu = yu_ref[...]
    shb = jnp.broadcast_to(sh[:, None, :], (SH_V, YU_V, SH_D)).reshape(
        TAB_ROWS, SH_D)
    yub = jnp.broadcast_to(yu[None, :, :], (SH_V, YU_V, YU_D)).reshape(
        TAB_ROWS, YU_D)
    out_ref[...] = jnp.concatenate([shb, yub], axis=-1)


def _build_table(sh_table, yu_table):
    return pl.pallas_call(
        _build_table_body,
        out_shape=jax.ShapeDtypeStruct((TAB_ROWS, OUT_D), jnp.float32),
    )(sh_table, yu_table)


def _make_sc_kernel(n_blocks):
    mesh = plsc.VectorSubcoreMesh(
        core_axis_name="c", subcore_axis_name="s",
        num_cores=NC, num_subcores=NS)

    @functools.partial(
        pl.kernel,
        out_type=jax.ShapeDtypeStruct((NW, n_blocks, BLK, OUT_D), jnp.float32),
        mesh=mesh,
        scratch_types=[
            pltpu.VMEM((n_blocks, BLK), jnp.int32),
            pltpu.VMEM((n_blocks, BLK), jnp.int32),
            [pltpu.VMEM((BLK, OUT_D), jnp.float32) for _ in range(NSLOT)],
            [pltpu.SemaphoreType.DMA for _ in range(NSLOT)],
            [pltpu.SemaphoreType.DMA for _ in range(NSLOT)],
            pltpu.VMEM_SHARED((TAB_ROWS, OUT_D), jnp.float32),
        ],
        compiler_params=pltpu.CompilerParams(use_tc_tiling_on_sc=False),
    )
    def sc_kernel(sidx_hbm, yidx_hbm, table_hbm, out_hbm,
                  comb_v, y_v, bufs, gsems, wsems, table_sh):
        sid = lax.axis_index("s")
        wid = sid * NC + lax.axis_index("c")

        @pl.when(sid == 0)
        def _stage_table():
            pltpu.sync_copy(table_hbm, table_sh)

        pltpu.sync_copy(sidx_hbm.at[wid], comb_v)
        pltpu.sync_copy(yidx_hbm.at[wid], y_v)
        plsc.subcore_barrier()

        def fuse(t, _):
            i = t // (BLK // L)
            j = (t % (BLK // L)) * L
            comb_v[i, pl.ds(j, L)] = (
                comb_v[i, pl.ds(j, L)] * YU_V + y_v[i, pl.ds(j, L)])
            return 0
        lax.fori_loop(0, n_blocks * (BLK // L), fuse, 0)

        for h in range(AHEAD):
            pltpu.async_copy(table_sh.at[comb_v.at[h]], bufs[h % NSLOT],
                             gsems[h % NSLOT])

        def step(t, _):
            for r in range(NSLOT):
                g = t * NSLOT + r
                pltpu.make_async_copy(table_sh.at[comb_v.at[g]], bufs[r],
                                      gsems[r]).wait()
                pltpu.async_copy(bufs[r], out_hbm.at[wid, g], wsems[r])
                h = g + AHEAD
                s = (r + AHEAD) % NSLOT

                @pl.when(h < n_blocks)
                def _prefetch():
                    @pl.when(h >= NSLOT)
                    def _reclaim():
                        pltpu.make_async_copy(
                            bufs[s], out_hbm.at[wid, h - NSLOT],
                            wsems[s]).wait()
                    pltpu.async_copy(table_sh.at[comb_v.at[h]], bufs[s],
                                     gsems[s])
            return 0

        lax.fori_loop(0, n_blocks // NSLOT, step, 0)

        for r in range(NSLOT):
            g = n_blocks - NSLOT + r
            pltpu.make_async_copy(bufs[r], out_hbm.at[wid, g],
                                  wsems[r]).wait()

    return sc_kernel


def kernel(shengmu_indices, yunmu_indices, shengmu_table, yunmu_table):
    batch, seq = shengmu_indices.shape
    n = batch * seq
    assert n % (NW * BLK * NSLOT) == 0
    n_blocks = n // (NW * BLK)

    table = _build_table(shengmu_table, yunmu_table)
    s = shengmu_indices.reshape(NW, n_blocks, BLK)
    y = yunmu_indices.reshape(NW, n_blocks, BLK)
    out = _make_sc_kernel(n_blocks)(s, y, table)
    return out.reshape(batch, seq, OUT_D)


# speedup vs baseline: 6.6936x; 1.7242x over previous
import functools

import jax
import jax.numpy as jnp
from jax import lax
from jax.experimental import pallas as pl
from jax.experimental.pallas import tpu as pltpu
from jax.experimental.pallas import tpu_sc as plsc

SH_V, YU_V = 24, 40
SH_D, YU_D = 32, 32
OUT_D = SH_D + YU_D
TAB_ROWS = SH_V * YU_V
NC, NS, L = 2, 16, 16
NW = NC * NS
BLK = 128
NSLOT = 4
AHEAD = 2


def _build_table_body(sh_ref, yu_ref, out_ref):
    sh = sh_ref[...]
    yu = yu_ref[...]
    shb = jnp.broadcast_to(sh[:, None, :], (SH_V, YU_V, SH_D)).reshape(
        TAB_ROWS, SH_D)
    yub = jnp.broadcast_to(yu[None, :, :], (SH_V, YU_V, YU_D)).reshape(
        TAB_ROWS, YU_D)
    out_ref[...] = jnp.concatenate([shb, yub], axis=-1)


def _build_table(sh_table, yu_table):
    return pl.pallas_call(
        _build_table_body,
        out_shape=jax.ShapeDtypeStruct((TAB_ROWS, OUT_D), jnp.float32),
    )(sh_table, yu_table)


def _make_sc_kernel(n_blocks):
    mesh = plsc.VectorSubcoreMesh(
        core_axis_name="c", subcore_axis_name="s",
        num_cores=NC, num_subcores=NS)

    @functools.partial(
        pl.kernel,
        out_type=jax.ShapeDtypeStruct((NW, n_blocks, BLK, OUT_D), jnp.float32),
        mesh=mesh,
        scratch_types=[
            pltpu.VMEM((n_blocks, BLK), jnp.int32),
            pltpu.VMEM((n_blocks, BLK), jnp.int32),
            [pltpu.VMEM((BLK, OUT_D), jnp.float32) for _ in range(NSLOT)],
            [pltpu.SemaphoreType.DMA for _ in range(NSLOT)],
            [pltpu.SemaphoreType.DMA for _ in range(NSLOT)],
            pltpu.VMEM_SHARED((TAB_ROWS, OUT_D), jnp.float32),
        ],
    )
    def sc_kernel(sidx_hbm, yidx_hbm, table_hbm, out_hbm,
                  comb_v, y_v, bufs, gsems, wsems, table_sh):
        sid = lax.axis_index("s")
        wid = sid * NC + lax.axis_index("c")

        @pl.when(sid == 0)
        def _stage_table():
            pltpu.sync_copy(table_hbm, table_sh)

        pltpu.sync_copy(sidx_hbm.at[wid], comb_v)
        pltpu.sync_copy(yidx_hbm.at[wid], y_v)
        plsc.subcore_barrier()

        def fuse(t, _):
            i = t // (BLK // L)
            j = (t % (BLK // L)) * L
            comb_v[i, pl.ds(j, L)] = (
                comb_v[i, pl.ds(j, L)] * YU_V + y_v[i, pl.ds(j, L)])
            return 0
        lax.fori_loop(0, n_blocks * (BLK // L), fuse, 0)

        for h in range(AHEAD):
            pltpu.async_copy(table_sh.at[comb_v.at[h]], bufs[h % NSLOT],
                             gsems[h % NSLOT])

        def step(t, _):
            for r in range(NSLOT):
                g = t * NSLOT + r
                pltpu.make_async_copy(table_sh.at[comb_v.at[g]], bufs[r],
                                      gsems[r]).wait()
                pltpu.async_copy(bufs[r], out_hbm.at[wid, g], wsems[r])
                h = g + AHEAD
                s = (r + AHEAD) % NSLOT

                @pl.when(h < n_blocks)
                def _prefetch():
                    @pl.when(h >= NSLOT)
                    def _reclaim():
                        pltpu.make_async_copy(
                            bufs[s], out_hbm.at[wid, h - NSLOT],
                            wsems[s]).wait()
                    pltpu.async_copy(table_sh.at[comb_v.at[h]], bufs[s],
                                     gsems[s])
            return 0

        lax.fori_loop(0, n_blocks // NSLOT, step, 0)

        for r in range(NSLOT):
            g = n_blocks - NSLOT + r
            pltpu.make_async_copy(bufs[r], out_hbm.at[wid, g],
                                  wsems[r]).wait()

    return sc_kernel


def kernel(shengmu_indices, yunmu_indices, shengmu_table, yunmu_table):
    batch, seq = shengmu_indices.shape
    n = batch * seq
    assert n % (NW * BLK * NSLOT) == 0
    n_blocks = n // (NW * BLK)

    table = _build_table(shengmu_table, yunmu_table)
    s = shengmu_indices.reshape(NW, n_blocks, BLK)
    y = yunmu_indices.reshape(NW, n_blocks, BLK)
    out = _make_sc_kernel(n_blocks)(s, y, table)
    return out.reshape(batch, seq, OUT_D)
